# Initial kernel scaffold; baseline (speedup 1.0000x reference)
import jax
import jax.numpy as jnp
from jax.experimental import pallas as pl


def kernel(params, wid, edge_index, graph_ids):
    raise NotImplementedError("write your pallas kernel here")


# speedup vs baseline: 3.1587x; 3.1587x over previous
import functools

import jax
import jax.numpy as jnp
from jax import lax
from jax.experimental import pallas as pl
from jax.experimental.pallas import tpu as pltpu

N = 10000
E = 320000
H = 128
V = 531
G = 128
HEADS = 4
CONVS = 2

BN = 128
N_PAD = 10240
NBLK = N_PAD // BN
E_PAD = 327680
C = 512
VP = 544
SENT = N_PAD
ER_ROWS = N_PAD + 128


def _embed_kernel(w1t_ref, emb_ref, w2t_ref, b_ref, wid_ref, gid_ref,
                  h0_ref, mean0_ref, table_ref, gsum_ref, gcnt_ref):
    nb = pl.program_id(0)

    @pl.when(nb == 0)
    def _():
        table_ref[...] = w1t_ref[...] + jnp.dot(
            emb_ref[...], w2t_ref[...], preferred_element_type=jnp.float32)
        gsum_ref[...] = jnp.zeros_like(gsum_ref)
        gcnt_ref[...] = jnp.zeros_like(gcnt_ref)

    wid = wid_ref[...]
    iota_v = lax.broadcasted_iota(jnp.int32, (BN, VP), 1).astype(jnp.float32)
    p = (wid == iota_v).astype(jnp.float32)
    h0 = jnp.dot(p, table_ref[...], preferred_element_type=jnp.float32)
    h0 = h0 + b_ref[...]
    h0_ref[...] = h0

    gid = gid_ref[...]
    iota_g = lax.broadcasted_iota(jnp.int32, (BN, G), 1).astype(jnp.float32)
    gp = (gid == iota_g).astype(jnp.float32)
    dn = (((0,), (0,)), ((), ()))
    gsum_ref[...] += lax.dot_general(gp, h0, dn,
                                     preferred_element_type=jnp.float32)
    gcnt_ref[...] += lax.dot_general(gp, jnp.ones((BN, H), jnp.float32), dn,
                                     preferred_element_type=jnp.float32)
    mean0_ref[...] = gsum_ref[...] / jnp.maximum(gcnt_ref[...], 1.0)


def _dense_kernel(h_ref, wlt_ref, al_ref, ar_ref, z_ref, el_ref, er_ref):
    nb = pl.program_id(0)
    z = jnp.dot(h_ref[...], wlt_ref[...], preferred_element_type=jnp.float32)
    z_ref[...] = z
    el_ref[...] = jnp.dot(z, al_ref[...], preferred_element_type=jnp.float32)
    er = jnp.dot(z, ar_ref[...], preferred_element_type=jnp.float32)
    rowid = (nb * BN
             + lax.broadcasted_iota(jnp.int32, (BN, 16), 0)).astype(jnp.float32)
    lane = lax.broadcasted_iota(jnp.int32, (BN, 16), 1)
    er_ref[...] = jnp.where(lane == 4, rowid, er)


def _gat_kernel(rowptr_ref, zs_ref, els_ref, erd_ref, hh_ref, stats_ref,
                zbuf, ebuf, rbuf, acc_ref, st_ref, sem0, sem1, sem2):
    nb = pl.program_id(0)
    acc_ref[...] = jnp.zeros_like(acc_ref)

    @pl.when(nb == 0)
    def _():
        st_ref[...] = jnp.zeros_like(st_ref)

    estart = (rowptr_ref[0, nb] // 8) * 8
    eend = rowptr_ref[0, nb + 1]
    nck = (eend - estart + C - 1) // C

    def body(c, carry):
        off = pl.multiple_of(estart + c * C, 8)
        cp0 = pltpu.make_async_copy(zs_ref.at[pl.ds(off, C)], zbuf, sem0)
        cp1 = pltpu.make_async_copy(els_ref.at[pl.ds(off, C)], ebuf, sem1)
        cp2 = pltpu.make_async_copy(erd_ref.at[pl.ds(off, C)], rbuf, sem2)
        cp0.start(); cp1.start(); cp2.start()
        cp0.wait(); cp1.wait(); cp2.wait()
        zs = zbuf[...]
        els = ebuf[...]
        erd = rbuf[...]
        dsv = erd[:, 4:5]
        iota_n = (nb * BN
                  + lax.broadcasted_iota(jnp.int32, (C, BN), 1)
                  ).astype(jnp.float32)
        p = (dsv == iota_n).astype(jnp.float32)
        vs = []
        exs = []
        for hd in range(HEADS):
            e = els[:, hd:hd + 1] + erd[:, hd:hd + 1]
            e = jnp.where(e >= 0.0, e, 0.01 * e)
            ex = jnp.exp(e)
            exs.append(ex)
            vs.append(zs[:, hd * H:(hd + 1) * H] * ex)
        v = jnp.concatenate(vs + exs + [jnp.zeros((C, 60), jnp.float32)],
                            axis=1)
        dn = (((0,), (0,)), ((), ()))
        acc_ref[...] += lax.dot_general(p, v, dn,
                                        preferred_element_type=jnp.float32)
        return carry

    lax.fori_loop(0, nck, body, 0)

    acc = acc_ref[...]
    hs = []
    for hd in range(HEADS):
        num = acc[:, hd * H:(hd + 1) * H]
        den = acc[:, 4 * H + hd:4 * H + hd + 1]
        hh = num / jnp.where(den > 0.0, den, 1.0)
        hs.append(jnp.maximum(hh, 0.0))
    hh_all = jnp.concatenate(hs, axis=1)
    hh_ref[...] = hh_all
    st_ref[0:1, :] += jnp.sum(hh_all, axis=0, keepdims=True)
    st_ref[1:2, :] += jnp.sum(hh_all * hh_all, axis=0, keepdims=True)
    stats_ref[...] = st_ref[...]


def _post_kernel(hh_ref, stats_ref, hprev_ref, wot_ref, b_ref, bng_ref,
                 bnb_ref, lng_ref, lnb_ref, gid_ref, h_ref, mean_ref,
                 gsum_ref, gcnt_ref):
    nb = pl.program_id(0)

    @pl.when(nb == 0)
    def _():
        gsum_ref[...] = jnp.zeros_like(gsum_ref)
        gcnt_ref[...] = jnp.zeros_like(gcnt_ref)

    x = hh_ref[...]
    mu = stats_ref[0:1, :] / float(N)
    var = stats_ref[1:2, :] / float(N) - mu * mu
    xn = bng_ref[...] * (x - mu) / jnp.sqrt(var + 1e-5) + bnb_ref[...]
    y = jnp.dot(xn, wot_ref[...], preferred_element_type=jnp.float32)
    y = y + b_ref[...] + hprev_ref[...]
    mu_r = jnp.mean(y, axis=1, keepdims=True)
    var_r = jnp.mean((y - mu_r) ** 2, axis=1, keepdims=True)
    h = lng_ref[...] * (y - mu_r) / jnp.sqrt(var_r + 1e-5) + lnb_ref[...]
    h_ref[...] = h

    gid = gid_ref[...]
    iota_g = lax.broadcasted_iota(jnp.int32, (BN, G), 1).astype(jnp.float32)
    gp = (gid == iota_g).astype(jnp.float32)
    dn = (((0,), (0,)), ((), ()))
    gsum_ref[...] += lax.dot_general(gp, h, dn,
                                     preferred_element_type=jnp.float32)
    gcnt_ref[...] += lax.dot_general(gp, jnp.ones((BN, H), jnp.float32), dn,
                                     preferred_element_type=jnp.float32)
    mean_ref[...] = gsum_ref[...] / jnp.maximum(gcnt_ref[...], 1.0)


def _gru_cell(x, h, wih_t, whh_t, bih, bhh):
    gx = jnp.dot(x, wih_t, preferred_element_type=jnp.float32) + bih
    gh = jnp.dot(h, whh_t, preferred_element_type=jnp.float32) + bhh
    xr, xz, xn = gx[:, :H], gx[:, H:2 * H], gx[:, 2 * H:]
    hr, hz, hn = gh[:, :H], gh[:, H:2 * H], gh[:, 2 * H:]
    r = jax.nn.sigmoid(xr + hr)
    z = jax.nn.sigmoid(xz + hz)
    n = jnp.tanh(xn + r * hn)
    return (1.0 - z) * n + z * h


def _gru_kernel(m0_ref, m1_ref, m2_ref,
                w00i, w00h, b00i, b00h, w01i, w01h, b01i, b01h,
                w10i, w10h, b10i, b10h, w11i, w11h, b11i, b11h, out_ref):
    xs = [m0_ref[...], m1_ref[...], m2_ref[...]]
    zero = jnp.zeros((G, H), jnp.float32)

    def run(xseq, wi, wh, bi, bh, reverse):
        hcur = zero
        outs = []
        order = (2, 1, 0) if reverse else (0, 1, 2)
        for t in order:
            hcur = _gru_cell(xseq[t], hcur, wi[...], wh[...], bi[...], bh[...])
            outs.append(hcur)
        if reverse:
            outs = outs[::-1]
        return outs, hcur

    o0f, h0f = run(xs, w00i, w00h, b00i, b00h, False)
    o0b, h0b = run(xs, w01i, w01h, b01i, b01h, True)
    x1 = [jnp.concatenate([o0f[t], o0b[t]], axis=1) for t in range(3)]
    _, h1f = run(x1, w10i, w10h, b10i, b10h, False)
    _, h1b = run(x1, w11i, w11h, b11i, b11h, True)
    out_ref[...] = (h0f + h0b + h1f + h1b) / 4.0


def _node_spec(width):
    return pl.BlockSpec((BN, width), lambda nb: (nb, 0))


def _full_spec(shape):
    return pl.BlockSpec(shape, lambda nb: tuple(0 for _ in shape))


def kernel(params, wid, edge_index, graph_ids):
    f32 = jnp.float32
    src = edge_index[0].astype(jnp.int32)
    dst = edge_index[1].astype(jnp.int32)

    perm = jnp.argsort(dst)
    ds = jnp.concatenate([dst[perm], jnp.full((E_PAD - E,), SENT, jnp.int32)])
    ss = jnp.concatenate([src[perm], jnp.zeros((E_PAD - E,), jnp.int32)])
    bounds = jnp.arange(NBLK + 1, dtype=jnp.int32) * BN
    rowptr = jnp.searchsorted(ds, bounds, side='left').astype(jnp.int32)
    rowptr = jnp.concatenate(
        [rowptr, jnp.zeros((128 - NBLK - 1,), jnp.int32)]).reshape(1, 128)

    wid_f = jnp.concatenate(
        [wid.astype(f32), jnp.zeros((N_PAD - N,), f32)]).reshape(N_PAD, 1)
    gid_f = jnp.concatenate(
        [graph_ids.astype(f32), jnp.full((N_PAD - N,), 999.0, f32)]
    ).reshape(N_PAD, 1)

    pw = params['proj_W']
    w1t = jnp.concatenate([pw[:, :V].T, jnp.zeros((VP - V, H), f32)], axis=0)
    emb_p = jnp.concatenate([params['emb'], jnp.zeros((VP - V, H), f32)],
                            axis=0)
    w2t = pw[:, V:].T
    pb = params['proj_b'].reshape(1, H)

    h0, mean0 = pl.pallas_call(
        _embed_kernel,
        grid=(NBLK,),
        in_specs=[_full_spec((VP, H)), _full_spec((VP, H)),
                  _full_spec((H, H)), _full_spec((1, H)),
                  _node_spec(1), _node_spec(1)],
        out_specs=[_node_spec(H), _full_spec((G, H))],
        out_shape=[jax.ShapeDtypeStruct((N_PAD, H), f32),
                   jax.ShapeDtypeStruct((G, H), f32)],
        scratch_shapes=[pltpu.VMEM((VP, H), f32), pltpu.VMEM((G, H), f32),
                        pltpu.VMEM((G, H), f32)],
    )(w1t, emb_p, w2t, pb, wid_f, gid_f)

    means = [mean0]
    h = h0
    for j in range(CONVS):
        gat = params['gat'][j]
        wlt = jnp.concatenate([gat[i]['Wl'].T for i in range(HEADS)], axis=1)
        al = jnp.zeros((HEADS * H, 16), f32)
        ar = jnp.zeros((HEADS * H, 16), f32)
        for i in range(HEADS):
            al = al.at[i * H:(i + 1) * H, i].set(gat[i]['Wa'][0, :H])
            ar = ar.at[i * H:(i + 1) * H, i].set(gat[i]['Wa'][0, H:])

        z_all, el16, er16 = pl.pallas_call(
            _dense_kernel,
            grid=(NBLK,),
            in_specs=[_node_spec(H), _full_spec((H, HEADS * H)),
                      _full_spec((HEADS * H, 16)), _full_spec((HEADS * H, 16))],
            out_specs=[_node_spec(HEADS * H), _node_spec(16), _node_spec(16)],
            out_shape=[jax.ShapeDtypeStruct((N_PAD, HEADS * H), f32),
                       jax.ShapeDtypeStruct((N_PAD, 16), f32),
                       jax.ShapeDtypeStruct((N_PAD, 16), f32)],
        )(h, wlt, al, ar)

        tail = jnp.where(
            lax.broadcasted_iota(jnp.int32, (ER_ROWS - N_PAD, 16), 1) == 4,
            N_PAD + lax.broadcasted_iota(f32, (ER_ROWS - N_PAD, 16), 0),
            jnp.where(
                lax.broadcasted_iota(jnp.int32, (ER_ROWS - N_PAD, 16), 1) < 4,
                -1e30, 0.0))
        er_tab = jnp.concatenate([er16, tail], axis=0)

        zs = jnp.take(z_all, ss, axis=0)
        els = jnp.take(el16, ss, axis=0)
        erd = jnp.take(er_tab, ds, axis=0)

        hh, stats = pl.pallas_call(
            _gat_kernel,
            grid=(NBLK,),
            in_specs=[pl.BlockSpec(memory_space=pltpu.MemorySpace.SMEM),
                      pl.BlockSpec(memory_space=pl.ANY),
                      pl.BlockSpec(memory_space=pl.ANY),
                      pl.BlockSpec(memory_space=pl.ANY)],
            out_specs=[_node_spec(HEADS * H), _full_spec((8, HEADS * H))],
            out_shape=[jax.ShapeDtypeStruct((N_PAD, HEADS * H), f32),
                       jax.ShapeDtypeStruct((8, HEADS * H), f32)],
            scratch_shapes=[pltpu.VMEM((C, HEADS * H), f32),
                            pltpu.VMEM((C, 16), f32),
                            pltpu.VMEM((C, 16), f32),
                            pltpu.VMEM((BN, 576), f32),
                            pltpu.VMEM((8, HEADS * H), f32),
                            pltpu.SemaphoreType.DMA,
                            pltpu.SemaphoreType.DMA,
                            pltpu.SemaphoreType.DMA],
        )(rowptr, zs, els, erd)

        wot = params['out'][j]['W'].T
        ob = params['out'][j]['b'].reshape(1, H)
        bng = jnp.concatenate([gat[i]['bn_g'] for i in range(HEADS)]).reshape(
            1, HEADS * H)
        bnb = jnp.concatenate([gat[i]['bn_b'] for i in range(HEADS)]).reshape(
            1, HEADS * H)
        lng = params['ln'][j]['g'].reshape(1, H)
        lnb = params['ln'][j]['b'].reshape(1, H)

        h, mean_j = pl.pallas_call(
            _post_kernel,
            grid=(NBLK,),
            in_specs=[_node_spec(HEADS * H), _full_spec((8, HEADS * H)),
                      _node_spec(H), _full_spec((HEADS * H, H)),
                      _full_spec((1, H)), _full_spec((1, HEADS * H)),
                      _full_spec((1, HEADS * H)), _full_spec((1, H)),
                      _full_spec((1, H)), _node_spec(1)],
            out_specs=[_node_spec(H), _full_spec((G, H))],
            out_shape=[jax.ShapeDtypeStruct((N_PAD, H), f32),
                       jax.ShapeDtypeStruct((G, H), f32)],
            scratch_shapes=[pltpu.VMEM((G, H), f32), pltpu.VMEM((G, H), f32)],
        )(hh, stats, h, wot, ob, bng, bnb, lng, lnb, gid_f)
        means.append(mean_j)

    gp = params['gru']
    gru_args = []
    gru_specs = []
    for l in range(2):
        for d in range(2):
            cell = gp[l][d]
            kin = H if l == 0 else 2 * H
            gru_args += [cell['Wih'].T, cell['Whh'].T,
                         cell['bih'].reshape(1, 3 * H),
                         cell['bhh'].reshape(1, 3 * H)]
            gru_specs += [_full_spec((kin, 3 * H)), _full_spec((H, 3 * H)),
                          _full_spec((1, 3 * H)), _full_spec((1, 3 * H))]

    out = pl.pallas_call(
        _gru_kernel,
        grid=(1,),
        in_specs=[_full_spec((G, H))] * 3 + gru_specs,
        out_specs=_full_spec((G, H)),
        out_shape=jax.ShapeDtypeStruct((G, H), f32),
    )(means[0], means[1], means[2], *gru_args)
    return out


# speedup vs baseline: 5.0709x; 1.6054x over previous
import functools

import jax
import jax.numpy as jnp
from jax import lax
from jax.experimental import pallas as pl
from jax.experimental.pallas import tpu as pltpu
from jax.experimental.pallas import tpu_sc as plsc

N = 10000
E = 320000
H = 128
V = 531
G = 128
HEADS = 4
CONVS = 2

BN = 128
N_PAD = 10240
NBLK = N_PAD // BN
E_PAD = 327680
C = 512
VP = 544
SENT = N_PAD
ER_ROWS = N_PAD + 128


NWORK = 32
EPW = E_PAD // NWORK


def _sc_gather(table, idx, width, chunk):
    mesh = plsc.VectorSubcoreMesh(core_axis_name="c", subcore_axis_name="s")

    @functools.partial(
        pl.kernel, mesh=mesh,
        out_type=jax.ShapeDtypeStruct((E_PAD, width), jnp.float32),
        scratch_types=[pltpu.VMEM((chunk,), jnp.int32),
                       pltpu.VMEM((chunk, width), jnp.float32),
                       pltpu.SemaphoreType.DMA],
    )
    def gather_k(tab_hbm, idx_hbm, out_hbm, idx_v, rows_v, sem):
        w = lax.axis_index("s") * 2 + lax.axis_index("c")
        base = w * EPW

        @pl.loop(0, EPW, step=chunk)
        def _(i):
            pltpu.sync_copy(idx_hbm.at[pl.ds(base + i, chunk)], idx_v)
            pltpu.async_copy(tab_hbm.at[idx_v], rows_v, sem).wait()
            pltpu.sync_copy(rows_v, out_hbm.at[pl.ds(base + i, chunk)])

    return gather_k(table, idx)


def _embed_kernel(w1t_ref, emb_ref, w2t_ref, b_ref, wid_ref, gid_ref,
                  h0_ref, mean0_ref, table_ref, gsum_ref, gcnt_ref):
    nb = pl.program_id(0)

    @pl.when(nb == 0)
    def _():
        table_ref[...] = w1t_ref[...] + jnp.dot(
            emb_ref[...], w2t_ref[...], preferred_element_type=jnp.float32)
        gsum_ref[...] = jnp.zeros_like(gsum_ref)
        gcnt_ref[...] = jnp.zeros_like(gcnt_ref)

    wid = wid_ref[...]
    iota_v = lax.broadcasted_iota(jnp.int32, (BN, VP), 1).astype(jnp.float32)
    p = (wid == iota_v).astype(jnp.float32)
    h0 = jnp.dot(p, table_ref[...], preferred_element_type=jnp.float32)
    h0 = h0 + b_ref[...]
    h0_ref[...] = h0

    gid = gid_ref[...]
    iota_g = lax.broadcasted_iota(jnp.int32, (BN, G), 1).astype(jnp.float32)
    gp = (gid == iota_g).astype(jnp.float32)
    dn = (((0,), (0,)), ((), ()))
    gsum_ref[...] += lax.dot_general(gp, h0, dn,
                                     preferred_element_type=jnp.float32)
    gcnt_ref[...] += lax.dot_general(gp, jnp.ones((BN, H), jnp.float32), dn,
                                     preferred_element_type=jnp.float32)
    mean0_ref[...] = gsum_ref[...] / jnp.maximum(gcnt_ref[...], 1.0)


def _dense_kernel(h_ref, wlt_ref, ar_ref, z_ref, er_ref):
    nb = pl.program_id(0)
    z = jnp.dot(h_ref[...], wlt_ref[...], preferred_element_type=jnp.float32)
    z_ref[...] = z
    er = jnp.dot(z, ar_ref[...], preferred_element_type=jnp.float32)
    er = jnp.concatenate([er, jnp.zeros((BN, 112), jnp.float32)], axis=1)
    rowid = (nb * BN
             + lax.broadcasted_iota(jnp.int32, (BN, 128), 0)).astype(jnp.float32)
    lane = lax.broadcasted_iota(jnp.int32, (BN, 128), 1)
    er_ref[...] = jnp.where(lane == 4, rowid, er)


def _gat_kernel(rowptr_ref, zs_ref, erd_ref, al_ref, hh_ref, stats_ref,
                zbuf, rbuf, acc_ref, st_ref, sem0, sem2):
    nb = pl.program_id(0)
    acc_ref[...] = jnp.zeros_like(acc_ref)

    @pl.when(nb == 0)
    def _():
        st_ref[...] = jnp.zeros_like(st_ref)

    estart = (rowptr_ref[0, nb] // 8) * 8
    eend = rowptr_ref[0, nb + 1]
    nck = (eend - estart + C - 1) // C

    def body(c, carry):
        off = pl.multiple_of(estart + c * C, 8)
        cp0 = pltpu.make_async_copy(zs_ref.at[pl.ds(off, C)], zbuf, sem0)
        cp2 = pltpu.make_async_copy(erd_ref.at[pl.ds(off, C)], rbuf, sem2)
        cp0.start(); cp2.start()
        cp0.wait(); cp2.wait()
        zs = zbuf[...]
        erd = rbuf[...]
        al = al_ref[...]
        dsv = erd[:, 4:5]
        iota_n = (nb * BN
                  + lax.broadcasted_iota(jnp.int32, (C, BN), 1)
                  ).astype(jnp.float32)
        p = (dsv == iota_n).astype(jnp.float32)
        vs = []
        exs = []
        for hd in range(HEADS):
            el = jnp.sum(zs[:, hd * H:(hd + 1) * H] * al[0:1, hd * H:(hd + 1) * H],
                         axis=1, keepdims=True)
            e = el + erd[:, hd:hd + 1]
            e = jnp.where(e >= 0.0, e, 0.01 * e)
            ex = jnp.exp(e)
            exs.append(ex)
            vs.append(zs[:, hd * H:(hd + 1) * H] * ex)
        v = jnp.concatenate(vs + exs + [jnp.zeros((C, 60), jnp.float32)],
                            axis=1)
        dn = (((0,), (0,)), ((), ()))
        acc_ref[...] += lax.dot_general(p, v, dn,
                                        preferred_element_type=jnp.float32)
        return carry

    lax.fori_loop(0, nck, body, 0)

    acc = acc_ref[...]
    hs = []
    for hd in range(HEADS):
        num = acc[:, hd * H:(hd + 1) * H]
        den = acc[:, 4 * H + hd:4 * H + hd + 1]
        hh = num / jnp.where(den > 0.0, den, 1.0)
        hs.append(jnp.maximum(hh, 0.0))
    hh_all = jnp.concatenate(hs, axis=1)
    hh_ref[...] = hh_all
    st_ref[0:1, :] += jnp.sum(hh_all, axis=0, keepdims=True)
    st_ref[1:2, :] += jnp.sum(hh_all * hh_all, axis=0, keepdims=True)
    stats_ref[...] = st_ref[...]


def _post_kernel(hh_ref, stats_ref, hprev_ref, wot_ref, b_ref, bng_ref,
                 bnb_ref, lng_ref, lnb_ref, gid_ref, h_ref, mean_ref,
                 gsum_ref, gcnt_ref):
    nb = pl.program_id(0)

    @pl.when(nb == 0)
    def _():
        gsum_ref[...] = jnp.zeros_like(gsum_ref)
        gcnt_ref[...] = jnp.zeros_like(gcnt_ref)

    x = hh_ref[...]
    mu = stats_ref[0:1, :] / float(N)
    var = stats_ref[1:2, :] / float(N) - mu * mu
    xn = bng_ref[...] * (x - mu) / jnp.sqrt(var + 1e-5) + bnb_ref[...]
    y = jnp.dot(xn, wot_ref[...], preferred_element_type=jnp.float32)
    y = y + b_ref[...] + hprev_ref[...]
    mu_r = jnp.mean(y, axis=1, keepdims=True)
    var_r = jnp.mean((y - mu_r) ** 2, axis=1, keepdims=True)
    h = lng_ref[...] * (y - mu_r) / jnp.sqrt(var_r + 1e-5) + lnb_ref[...]
    h_ref[...] = h

    gid = gid_ref[...]
    iota_g = lax.broadcasted_iota(jnp.int32, (BN, G), 1).astype(jnp.float32)
    gp = (gid == iota_g).astype(jnp.float32)
    dn = (((0,), (0,)), ((), ()))
    gsum_ref[...] += lax.dot_general(gp, h, dn,
                                     preferred_element_type=jnp.float32)
    gcnt_ref[...] += lax.dot_general(gp, jnp.ones((BN, H), jnp.float32), dn,
                                     preferred_element_type=jnp.float32)
    mean_ref[...] = gsum_ref[...] / jnp.maximum(gcnt_ref[...], 1.0)


def _gru_cell(x, h, wih_t, whh_t, bih, bhh):
    gx = jnp.dot(x, wih_t, preferred_element_type=jnp.float32) + bih
    gh = jnp.dot(h, whh_t, preferred_element_type=jnp.float32) + bhh
    xr, xz, xn = gx[:, :H], gx[:, H:2 * H], gx[:, 2 * H:]
    hr, hz, hn = gh[:, :H], gh[:, H:2 * H], gh[:, 2 * H:]
    r = jax.nn.sigmoid(xr + hr)
    z = jax.nn.sigmoid(xz + hz)
    n = jnp.tanh(xn + r * hn)
    return (1.0 - z) * n + z * h


def _gru_kernel(m0_ref, m1_ref, m2_ref,
                w00i, w00h, b00i, b00h, w01i, w01h, b01i, b01h,
                w10i, w10h, b10i, b10h, w11i, w11h, b11i, b11h, out_ref):
    xs = [m0_ref[...], m1_ref[...], m2_ref[...]]
    zero = jnp.zeros((G, H), jnp.float32)

    def run(xseq, wi, wh, bi, bh, reverse):
        hcur = zero
        outs = []
        order = (2, 1, 0) if reverse else (0, 1, 2)
        for t in order:
            hcur = _gru_cell(xseq[t], hcur, wi[...], wh[...], bi[...], bh[...])
            outs.append(hcur)
        if reverse:
            outs = outs[::-1]
        return outs, hcur

    o0f, h0f = run(xs, w00i, w00h, b00i, b00h, False)
    o0b, h0b = run(xs, w01i, w01h, b01i, b01h, True)
    x1 = [jnp.concatenate([o0f[t], o0b[t]], axis=1) for t in range(3)]
    _, h1f = run(x1, w10i, w10h, b10i, b10h, False)
    _, h1b = run(x1, w11i, w11h, b11i, b11h, True)
    out_ref[...] = (h0f + h0b + h1f + h1b) / 4.0


def _node_spec(width):
    return pl.BlockSpec((BN, width), lambda nb: (nb, 0))


def _full_spec(shape):
    return pl.BlockSpec(shape, lambda nb: tuple(0 for _ in shape))


def kernel(params, wid, edge_index, graph_ids):
    f32 = jnp.float32
    src = edge_index[0].astype(jnp.int32)
    dst = edge_index[1].astype(jnp.int32)

    perm = jnp.argsort(dst)
    ds = jnp.concatenate([dst[perm], jnp.full((E_PAD - E,), SENT, jnp.int32)])
    ss = jnp.concatenate([src[perm], jnp.zeros((E_PAD - E,), jnp.int32)])
    bounds = jnp.arange(NBLK + 1, dtype=jnp.int32) * BN
    rowptr = jnp.searchsorted(ds, bounds, side='left').astype(jnp.int32)
    rowptr = jnp.concatenate(
        [rowptr, jnp.zeros((128 - NBLK - 1,), jnp.int32)]).reshape(1, 128)

    wid_f = jnp.concatenate(
        [wid.astype(f32), jnp.zeros((N_PAD - N,), f32)]).reshape(N_PAD, 1)
    gid_f = jnp.concatenate(
        [graph_ids.astype(f32), jnp.full((N_PAD - N,), 999.0, f32)]
    ).reshape(N_PAD, 1)

    pw = params['proj_W']
    w1t = jnp.concatenate([pw[:, :V].T, jnp.zeros((VP - V, H), f32)], axis=0)
    emb_p = jnp.concatenate([params['emb'], jnp.zeros((VP - V, H), f32)],
                            axis=0)
    w2t = pw[:, V:].T
    pb = params['proj_b'].reshape(1, H)

    h0, mean0 = pl.pallas_call(
        _embed_kernel,
        grid=(NBLK,),
        in_specs=[_full_spec((VP, H)), _full_spec((VP, H)),
                  _full_spec((H, H)), _full_spec((1, H)),
                  _node_spec(1), _node_spec(1)],
        out_specs=[_node_spec(H), _full_spec((G, H))],
        out_shape=[jax.ShapeDtypeStruct((N_PAD, H), f32),
                   jax.ShapeDtypeStruct((G, H), f32)],
        scratch_shapes=[pltpu.VMEM((VP, H), f32), pltpu.VMEM((G, H), f32),
                        pltpu.VMEM((G, H), f32)],
    )(w1t, emb_p, w2t, pb, wid_f, gid_f)

    means = [mean0]
    h = h0
    for j in range(CONVS):
        gat = params['gat'][j]
        wlt = jnp.concatenate([gat[i]['Wl'].T for i in range(HEADS)], axis=1)
        ar = jnp.zeros((HEADS * H, 16), f32)
        for i in range(HEADS):
            ar = ar.at[i * H:(i + 1) * H, i].set(gat[i]['Wa'][0, H:])

        z_all, er128 = pl.pallas_call(
            _dense_kernel,
            grid=(NBLK,),
            in_specs=[_node_spec(H), _full_spec((H, HEADS * H)),
                      _full_spec((HEADS * H, 16))],
            out_specs=[_node_spec(HEADS * H), _node_spec(128)],
            out_shape=[jax.ShapeDtypeStruct((N_PAD, HEADS * H), f32),
                       jax.ShapeDtypeStruct((N_PAD, 128), f32)],
        )(h, wlt, ar)

        tail = jnp.where(
            lax.broadcasted_iota(jnp.int32, (ER_ROWS - N_PAD, 128), 1) == 4,
            N_PAD + lax.broadcasted_iota(f32, (ER_ROWS - N_PAD, 128), 0),
            jnp.where(
                lax.broadcasted_iota(jnp.int32, (ER_ROWS - N_PAD, 128), 1) < 4,
                -1e30, 0.0))
        er_tab = jnp.concatenate([er128, tail], axis=0)

        zs = _sc_gather(z_all, ss, HEADS * H, 128)
        erd = _sc_gather(er_tab, ds, 128, 512)
        al_tab = jnp.concatenate(
            [gat[i]['Wa'][0, :H] for i in range(HEADS)]).reshape(1, HEADS * H)

        hh, stats = pl.pallas_call(
            _gat_kernel,
            grid=(NBLK,),
            in_specs=[pl.BlockSpec(memory_space=pltpu.MemorySpace.SMEM),
                      pl.BlockSpec(memory_space=pl.ANY),
                      pl.BlockSpec(memory_space=pl.ANY),
                      _full_spec((1, HEADS * H))],
            out_specs=[_node_spec(HEADS * H), _full_spec((8, HEADS * H))],
            out_shape=[jax.ShapeDtypeStruct((N_PAD, HEADS * H), f32),
                       jax.ShapeDtypeStruct((8, HEADS * H), f32)],
            scratch_shapes=[pltpu.VMEM((C, HEADS * H), f32),
                            pltpu.VMEM((C, 128), f32),
                            pltpu.VMEM((BN, 576), f32),
                            pltpu.VMEM((8, HEADS * H), f32),
                            pltpu.SemaphoreType.DMA,
                            pltpu.SemaphoreType.DMA],
        )(rowptr, zs, erd, al_tab)

        wot = params['out'][j]['W'].T
        ob = params['out'][j]['b'].reshape(1, H)
        bng = jnp.concatenate([gat[i]['bn_g'] for i in range(HEADS)]).reshape(
            1, HEADS * H)
        bnb = jnp.concatenate([gat[i]['bn_b'] for i in range(HEADS)]).reshape(
            1, HEADS * H)
        lng = params['ln'][j]['g'].reshape(1, H)
        lnb = params['ln'][j]['b'].reshape(1, H)

        h, mean_j = pl.pallas_call(
            _post_kernel,
            grid=(NBLK,),
            in_specs=[_node_spec(HEADS * H), _full_spec((8, HEADS * H)),
                      _node_spec(H), _full_spec((HEADS * H, H)),
                      _full_spec((1, H)), _full_spec((1, HEADS * H)),
                      _full_spec((1, HEADS * H)), _full_spec((1, H)),
                      _full_spec((1, H)), _node_spec(1)],
            out_specs=[_node_spec(H), _full_spec((G, H))],
            out_shape=[jax.ShapeDtypeStruct((N_PAD, H), f32),
                       jax.ShapeDtypeStruct((G, H), f32)],
            scratch_shapes=[pltpu.VMEM((G, H), f32), pltpu.VMEM((G, H), f32)],
        )(hh, stats, h, wot, ob, bng, bnb, lng, lnb, gid_f)
        means.append(mean_j)

    gp = params['gru']
    gru_args = []
    gru_specs = []
    for l in range(2):
        for d in range(2):
            cell = gp[l][d]
            kin = H if l == 0 else 2 * H
            gru_args += [cell['Wih'].T, cell['Whh'].T,
                         cell['bih'].reshape(1, 3 * H),
                         cell['bhh'].reshape(1, 3 * H)]
            gru_specs += [_full_spec((kin, 3 * H)), _full_spec((H, 3 * H)),
                          _full_spec((1, 3 * H)), _full_spec((1, 3 * H))]

    out = pl.pallas_call(
        _gru_kernel,
        grid=(1,),
        in_specs=[_full_spec((G, H))] * 3 + gru_specs,
        out_specs=_full_spec((G, H)),
        out_shape=jax.ShapeDtypeStruct((G, H), f32),
    )(means[0], means[1], means[2], *gru_args)
    return out
